# baseline (speedup 1.0000x reference)
import jax
import jax.numpy as jnp
from jax import lax
from jax.experimental import pallas as pl

_B, _C, _H, _W = 8, 21, 512, 512
_HW = _H * _W
_EPS = 1e-05
_K = 512
_CHUNK = _K * 128
_NCHUNK = _HW // _CHUNK


def _dice_sums_body(pred_ref, tgt_ref, out_ref):
    j = pl.program_id(1)
    cls = lax.broadcasted_iota(jnp.int32, (_C, 128), 0)

    def step(k, accs):
        ia, p2a, ca = accs
        p = pred_ref[0, :, k, :]
        t = tgt_ref[0, :, k, :]
        m = cls == t
        ia = ia + jnp.where(m, p, 0.0)
        p2a = p2a + p * p
        ca = ca + jnp.where(m, 1.0, 0.0)
        return ia, p2a, ca

    zero = jnp.zeros((_C, 128), jnp.float32)
    ia, p2a, ca = lax.fori_loop(0, _K, step, (zero, zero, zero))
    part = jnp.concatenate([ia, p2a, ca], axis=0)

    @pl.when(j == 0)
    def _():
        out_ref[0] = part

    @pl.when(j != 0)
    def _():
        out_ref[0] += part


def kernel(prediction, target):
    pred4 = prediction.reshape(_B, _C, _HW // 128, 128)
    tgt4 = target.astype(jnp.int32).reshape(_B, 1, _HW // 128, 128)

    sums = pl.pallas_call(
        _dice_sums_body,
        grid=(_B, _NCHUNK),
        in_specs=[
            pl.BlockSpec((1, _C, _K, 128), lambda b, j: (b, 0, j, 0)),
            pl.BlockSpec((1, 1, _K, 128), lambda b, j: (b, 0, j, 0)),
        ],
        out_specs=pl.BlockSpec((1, 3 * _C, 128), lambda b, j: (b, 0, 0)),
        out_shape=jax.ShapeDtypeStruct((_B, 3 * _C, 128), jnp.float32),
    )(pred4, tgt4)

    sums = sums.sum(axis=-1)
    inter = sums[:, :_C]
    p2 = sums[:, _C:2 * _C]
    cnt = sums[:, 2 * _C:]
    dice = (2.0 * inter + _EPS) / (p2 + cnt + _EPS)
    return 1.0 - dice.mean()


# speedup vs baseline: 1.9924x; 1.9924x over previous
import jax
import jax.numpy as jnp
from jax import lax
from jax.experimental import pallas as pl

_B, _C, _H, _W = 8, 21, 512, 512
_HW = _H * _W
_EPS = 1e-05
_G = 3
_CG = _C // _G
_ROWS = _HW // 1024
_KJ = 64
_NJ = _ROWS // _KJ


def _dice_sums_body(pred_ref, tgt_ref, out_ref):
    g = pl.program_id(1)
    j = pl.program_id(2)
    cid = g * _CG + lax.broadcasted_iota(jnp.int32, (_CG, 1, 1), 0)

    def step(k, accs):
        ia, p2a, ca = accs
        p = pred_ref[0, :, k, :, :]
        t = tgt_ref[0, 0, k, :, :]
        m = t[None, :, :] == cid
        ia = ia + jnp.where(m, p, 0.0)
        p2a = p2a + p * p
        ca = ca + jnp.where(m, 1.0, 0.0)
        return ia, p2a, ca

    zero = jnp.zeros((_CG, 8, 128), jnp.float32)
    ia, p2a, ca = lax.fori_loop(0, _KJ, step, (zero, zero, zero), unroll=2)
    part = jnp.concatenate([ia, p2a, ca], axis=0)
    part = part.reshape(3 * _CG * 8, 128)

    @pl.when(j == 0)
    def _():
        out_ref[0, 0] = part

    @pl.when(j != 0)
    def _():
        out_ref[0, 0] += part


def kernel(prediction, target):
    pred5 = prediction.reshape(_B * _G, _CG, _ROWS, 8, 128)
    tgt5 = target.astype(jnp.int32).reshape(_B, 1, _ROWS, 8, 128)

    sums = pl.pallas_call(
        _dice_sums_body,
        grid=(_B, _G, _NJ),
        in_specs=[
            pl.BlockSpec((1, _CG, _KJ, 8, 128),
                         lambda b, g, j: (b * _G + g, 0, j, 0, 0)),
            pl.BlockSpec((1, 1, _KJ, 8, 128),
                         lambda b, g, j: (b, 0, j, 0, 0)),
        ],
        out_specs=pl.BlockSpec((1, 1, 3 * _CG * 8, 128),
                               lambda b, g, j: (b, g, 0, 0)),
        out_shape=jax.ShapeDtypeStruct((_B, _G, 3 * _CG * 8, 128),
                                       jnp.float32),
    )(pred5, tgt5)

    s = sums.reshape(_B, _G, 3, _CG, 8, 128).sum(axis=(4, 5))
    s = s.transpose(0, 2, 1, 3).reshape(_B, 3, _C)
    inter, p2, cnt = s[:, 0], s[:, 1], s[:, 2]
    dice = (2.0 * inter + _EPS) / (p2 + cnt + _EPS)
    return 1.0 - dice.mean()
